# Initial kernel scaffold; baseline (speedup 1.0000x reference)
import functools

import jax
import jax.numpy as jnp
from jax import lax
from jax.experimental import pallas as pl
from jax.experimental.pallas import tpu as pltpu
from jax.experimental.pallas import tpu_sc as plsc

NC, NS = 2, 16
NW = NC * NS
CHUNK = 128


@functools.partial(jax.jit, static_argnums=(2, 3))
def _sc_gather(idx2d, table, B, D):
    n_rows = idx2d.shape[0]
    rows_per_w = n_rows // NW
    b_per_w = B // NW
    mesh = plsc.VectorSubcoreMesh(core_axis_name="c", subcore_axis_name="s")

    @functools.partial(
        pl.kernel,
        out_type=jax.ShapeDtypeStruct((B, D), jnp.float32),
        mesh=mesh,
        scratch_types=[
            pltpu.VMEM((rows_per_w, CHUNK), jnp.int32),
            pltpu.VMEM((CHUNK, D), jnp.float32),
            pltpu.SemaphoreType.DMA,
        ],
    )
    def k(idx_hbm, table_hbm, out_hbm, idx_v, rows_v, sem):
        wid = lax.axis_index("s") * NC + lax.axis_index("c")
        row0 = wid * rows_per_w
        base = wid * b_per_w
        pltpu.sync_copy(idx_hbm.at[pl.ds(row0, rows_per_w)], idx_v)

        def body(j, carry):
            pltpu.async_copy(table_hbm.at[idx_v.at[j]], rows_v, sem).wait()
            pltpu.sync_copy(rows_v, out_hbm.at[pl.ds(base + j * CHUNK, CHUNK)])
            return carry

        lax.fori_loop(0, rows_per_w, body, 0)

    return k(idx2d, table)


def kernel(token_ids, embedding_lookup):
    s0, s1 = token_ids.shape
    B = s0 * s1
    D = embedding_lookup.shape[1]
    idx2d = token_ids.reshape(B // CHUNK, CHUNK).astype(jnp.int32)
    out = _sc_gather(idx2d, embedding_lookup, B, D)
    return out.reshape(s0, s1, D)


# speedup vs baseline: 1.6843x; 1.6843x over previous
import functools

import jax
import jax.numpy as jnp
from jax import lax
from jax.experimental import pallas as pl
from jax.experimental.pallas import tpu as pltpu
from jax.experimental.pallas import tpu_sc as plsc

NC, NS = 2, 16
NW = NC * NS
CHUNK = 128


@functools.partial(jax.jit, static_argnums=(2, 3))
def _sc_gather(idx2d, table, B, D):
    n_rows = idx2d.shape[0]
    rows_per_w = n_rows // NW
    b_per_w = B // NW
    mesh = plsc.VectorSubcoreMesh(core_axis_name="c", subcore_axis_name="s")

    @functools.partial(
        pl.kernel,
        out_type=jax.ShapeDtypeStruct((B, D), jnp.float32),
        mesh=mesh,
        scratch_types=[
            pltpu.VMEM((rows_per_w, CHUNK), jnp.int32),
            pltpu.VMEM((CHUNK, D), jnp.float32),
            pltpu.SemaphoreType.DMA,
        ],
        compiler_params=pltpu.CompilerParams(use_tc_tiling_on_sc=False),
    )
    def k(idx_hbm, table_hbm, out_hbm, idx_v, rows_v, sem):
        wid = lax.axis_index("s") * NC + lax.axis_index("c")
        row0 = wid * rows_per_w
        base = wid * b_per_w
        pltpu.sync_copy(idx_hbm.at[pl.ds(row0, rows_per_w)], idx_v)

        def body(j, carry):
            pltpu.async_copy(table_hbm.at[idx_v.at[j]], rows_v, sem).wait()
            pltpu.sync_copy(rows_v, out_hbm.at[pl.ds(base + j * CHUNK, CHUNK)])
            return carry

        lax.fori_loop(0, rows_per_w, body, 0)

    return k(idx2d, table)


def kernel(token_ids, embedding_lookup):
    s0, s1 = token_ids.shape
    B = s0 * s1
    D = embedding_lookup.shape[1]
    idx2d = token_ids.reshape(B // CHUNK, CHUNK).astype(jnp.int32)
    out = _sc_gather(idx2d, embedding_lookup, B, D)
    return out.reshape(s0, s1, D)


# speedup vs baseline: 1.8725x; 1.1117x over previous
import functools

import jax
import jax.numpy as jnp
from jax import lax
from jax.experimental import pallas as pl
from jax.experimental.pallas import tpu as pltpu
from jax.experimental.pallas import tpu_sc as plsc

NC, NS = 2, 16
NW = NC * NS
CHUNK = 128
NBUF = 8


@functools.partial(jax.jit, static_argnums=(2, 3))
def _sc_gather(idx2d, table, B, D):
    n_rows = idx2d.shape[0]
    rows_per_w = n_rows // NW
    b_per_w = B // NW
    assert rows_per_w % NBUF == 0 and rows_per_w // NBUF >= 2
    n_grps = rows_per_w // NBUF
    mesh = plsc.VectorSubcoreMesh(core_axis_name="c", subcore_axis_name="s")

    @functools.partial(
        pl.kernel,
        out_type=jax.ShapeDtypeStruct((B, D), jnp.float32),
        mesh=mesh,
        scratch_types=[
            pltpu.VMEM((rows_per_w, CHUNK), jnp.int32),
            pltpu.VMEM((NBUF, CHUNK, D), jnp.float32),
            pltpu.SemaphoreType.DMA,
            pltpu.SemaphoreType.DMA,
        ],
        compiler_params=pltpu.CompilerParams(use_tc_tiling_on_sc=False),
    )
    def k(idx_hbm, table_hbm, out_hbm, idx_v, bufs, sem_g, sem_w):
        wid = lax.axis_index("s") * NC + lax.axis_index("c")
        row0 = wid * rows_per_w
        base = wid * b_per_w
        pltpu.sync_copy(idx_hbm.at[pl.ds(row0, rows_per_w)], idx_v)

        def start_gather(j, b):
            pltpu.async_copy(table_hbm.at[idx_v.at[j]], bufs.at[b], sem_g)

        def wait_gather(b):
            pltpu.make_async_copy(
                table_hbm.at[idx_v.at[0]], bufs.at[b], sem_g).wait()

        def start_write(j, b):
            pltpu.async_copy(
                bufs.at[b], out_hbm.at[pl.ds(base + j * CHUNK, CHUNK)], sem_w)

        def wait_write(b):
            pltpu.make_async_copy(
                bufs.at[b], out_hbm.at[pl.ds(base, CHUNK)], sem_w).wait()

        for b in range(NBUF):
            start_gather(b, b)

        def group(grp, carry):
            j0 = grp * NBUF
            for b in range(NBUF):
                wait_gather(b)
                start_write(j0 + b, b)
            for b in range(NBUF):
                wait_write(b)
                start_gather(j0 + NBUF + b, b)
            return carry

        lax.fori_loop(0, n_grps - 1, group, 0, unroll=False)

        j0 = (n_grps - 1) * NBUF
        for b in range(NBUF):
            wait_gather(b)
            start_write(j0 + b, b)
        for b in range(NBUF):
            wait_write(b)

    return k(idx2d, table)


def kernel(token_ids, embedding_lookup):
    s0, s1 = token_ids.shape
    B = s0 * s1
    D = embedding_lookup.shape[1]
    idx2d = token_ids.reshape(B // CHUNK, CHUNK).astype(jnp.int32)
    out = _sc_gather(idx2d, embedding_lookup, B, D)
    return out.reshape(s0, s1, D)
